# baseline (speedup 1.0000x reference)
import functools

import jax
import jax.numpy as jnp
from jax import lax
from jax.experimental import pallas as pl
from jax.experimental.pallas import tpu as pltpu
from jax.experimental.pallas import tpu_sc as plsc

_NCLS = 30
_HI = 799.0
_G = 8
_NP = 5120
_D = 2048
_NC = _NP // 16
_NW = _NC // 16


def _clip(v):
    return jnp.clip(v, 1.0, _HI)


def _head_body(gt_ref, lab_ref, p8_ref, props_ref, ov_ref, tm_ref, mask_ref, f_ref):
    px1 = _clip(props_ref[0:1, :])
    py1 = _clip(props_ref[1:2, :])
    px2 = _clip(props_ref[2:3, :])
    py2 = _clip(props_ref[3:4, :])
    area = (px2 - px1 + 1.0) * (py2 - py1 + 1.0)

    ious = []
    for g in range(_G):
        gx1 = _clip(gt_ref[g, 0])
        gy1 = _clip(gt_ref[g, 1])
        gx2 = _clip(gt_ref[g, 2])
        gy2 = _clip(gt_ref[g, 3])
        iw = jnp.maximum(jnp.minimum(px2, gx2) - jnp.maximum(px1, gx1) + 1.0, 0.0)
        ih = jnp.maximum(jnp.minimum(py2, gy2) - jnp.maximum(py1, gy1) + 1.0, 0.0)
        inter = iw * ih
        ag = (gx2 - gx1 + 1.0) * (gy2 - gy1 + 1.0)
        ious.append(inter / (area + ag - inter))

    iota_c = jax.lax.broadcasted_iota(jnp.int32, (_NCLS, 1), 0)
    ov = jnp.zeros((_NCLS, _NP), jnp.float32)
    for g in range(_G):
        sel = iota_c == lab_ref[g]
        ov = jnp.maximum(ov, jnp.where(sel, ious[g], 0.0))
    ov_ref[...] = ov

    masks = []
    for g in range(_G):
        ol = ious[g]
        for g2 in range(_G):
            if g2 != g:
                same = lab_ref[g] == lab_ref[g2]
                ol = jnp.maximum(ol, jnp.where(same, ious[g2], 0.0))
        masks.append((ol > 0.6).astype(jnp.float32))
    mask = jnp.concatenate(masks, axis=0)
    mask_ref[...] = mask

    f = jnp.max(mask.reshape(_G, _NC, 16), axis=2)
    f_ref[...] = jnp.concatenate([f, jnp.zeros((_G, 384 - _NC), jnp.float32)], axis=1)

    src_w = px2 - px1
    src_h = py2 - py1
    src_cx = px1 + 0.5 * src_w
    src_cy = py1 + 0.5 * src_h
    for g in range(_G):
        q1 = _clip(p8_ref[g, 0])
        q2 = _clip(p8_ref[g, 1])
        q3 = _clip(p8_ref[g, 2])
        q4 = _clip(p8_ref[g, 3])
        gw = q3 - q1
        gh = q4 - q2
        gcx = q1 + 0.5 * gw
        gcy = q2 + 0.5 * gh
        dcx = (gcx - src_cx) / src_w
        dcy = (gcy - src_cy) / src_h
        dw = jnp.log(gw / src_w)
        dh = jnp.log(gh / src_h)
        t4 = jnp.concatenate([dcx, dcy, dw, dh], axis=0) * masks[g]
        tm_ref[:, g, :] = t4


def _sc_body(mask_hbm, f_hbm, x_hbm, out_hbm, m_v, f_v, idx_v, rows_v, acc_v, cnt_s, sem):
    wid = lax.axis_index("s") * 2 + lax.axis_index("c")

    @pl.when(wid < _G)
    def _():
        g = wid
        lane = lax.broadcasted_iota(jnp.int32, (16,), 0)
        zeros16 = jnp.zeros((16,), jnp.float32)
        pltpu.sync_copy(mask_hbm.at[g], m_v)
        pltpu.sync_copy(f_hbm.at[g], f_v)
        cnt_s[0] = 0

        def append(n):
            w = cnt_s[0]
            base = (w // 16) * 16
            r = w - base

            @pl.when(r == 0)
            def _():
                idx_v[pl.ds(base, 16)] = jnp.where(lane == 0, n, 0)

            @pl.when(r > 0)
            def _():
                ivv = idx_v[pl.ds(base, 16)]
                idx_v[pl.ds(base, 16)] = jnp.where(lane == r, n, ivv)

            cnt_s[0] = w + 1

        def word_body(wb, carry):
            fw = f_v[pl.ds(wb * 16, 16)]
            anyf = fw[0]
            for fj in range(1, 16):
                anyf = jnp.maximum(anyf, fw[fj])

            @pl.when(anyf > 0.5)
            def _():
                for fj in range(16):

                    @pl.when(fw[fj] > 0.5)
                    def _():
                        c = wb * 16 + fj
                        mvec = m_v[pl.ds(c * 16, 16)]
                        for l in range(16):

                            @pl.when(mvec[l] > 0.5)
                            def _():
                                append(c * 16 + l)

            return carry

        lax.fori_loop(0, _NW, word_body, 0)
        total = cnt_s[0]

        for k in range(_D // 16):
            acc_v[pl.ds(k * 16, 16)] = zeros16

        def batch_body(b, carry):
            pltpu.async_copy(x_hbm.at[idx_v.at[pl.ds(b * 16, 16)]], rows_v, sem).wait()
            rem = jnp.minimum(total - b * 16, 16)

            def row_body(l, carry2):
                for k in range(_D // 16):
                    acc_v[pl.ds(k * 16, 16)] += rows_v[l, pl.ds(k * 16, 16)]
                return carry2

            lax.fori_loop(0, rem, row_body, 0)
            return carry

        nb = (total + 15) // 16
        lax.fori_loop(0, nb, batch_body, 0)
        pltpu.sync_copy(acc_v, out_hbm.at[g])


def _posfeat_sc(mask_pad, flags, x):
    scmesh = plsc.VectorSubcoreMesh(core_axis_name="c", subcore_axis_name="s")
    f = functools.partial(
        pl.kernel,
        out_type=jax.ShapeDtypeStruct((_G, _D), jnp.float32),
        mesh=scmesh,
        scratch_types=[
            pltpu.VMEM((_NP,), jnp.float32),
            pltpu.VMEM((384,), jnp.float32),
            pltpu.VMEM((_NP + 16,), jnp.int32),
            pltpu.VMEM((16, _D), jnp.float32),
            pltpu.VMEM((_D,), jnp.float32),
            pltpu.SMEM((8,), jnp.int32),
            pltpu.SemaphoreType.DMA,
        ],
    )(_sc_body)
    return f(mask_pad, flags, x)


def kernel(x, proposals, gt_bbox, gt_labels):
    n, d = x.shape
    props_t = jnp.concatenate(
        [proposals.T, jnp.zeros((4, _NP - n), jnp.float32)], axis=1
    )
    p8 = proposals[:_G]

    ov_cn, tm, mask_pad, flags = pl.pallas_call(
        _head_body,
        in_specs=[
            pl.BlockSpec(memory_space=pltpu.SMEM),
            pl.BlockSpec(memory_space=pltpu.SMEM),
            pl.BlockSpec(memory_space=pltpu.SMEM),
            pl.BlockSpec((4, _NP), lambda: (0, 0)),
        ],
        out_specs=[
            pl.BlockSpec((_NCLS, _NP), lambda: (0, 0)),
            pl.BlockSpec((4, _G, _NP), lambda: (0, 0, 0)),
            pl.BlockSpec((_G, _NP), lambda: (0, 0)),
            pl.BlockSpec((_G, 384), lambda: (0, 0)),
        ],
        out_shape=[
            jax.ShapeDtypeStruct((_NCLS, _NP), jnp.float32),
            jax.ShapeDtypeStruct((4, _G, _NP), jnp.float32),
            jax.ShapeDtypeStruct((_G, _NP), jnp.float32),
            jax.ShapeDtypeStruct((_G, 384), jnp.float32),
        ],
    )(gt_bbox, gt_labels.astype(jnp.int32), p8, props_t)

    pf = _posfeat_sc(mask_pad, flags, x)
    return ov_cn[:, :n].T, tm[:, :, :n].transpose(1, 2, 0), pf


# speedup vs baseline: 1.1442x; 1.1442x over previous
import functools

import jax
import jax.numpy as jnp
from jax import lax
from jax.experimental import pallas as pl
from jax.experimental.pallas import tpu as pltpu
from jax.experimental.pallas import tpu_sc as plsc

_NCLS = 30
_HI = 799.0
_G = 8
_NP = 5120
_D = 2048
_NC = _NP // 16
_NW = _NC // 16


def _clip(v):
    return jnp.clip(v, 1.0, _HI)


def _head_body(gt_ref, lab_ref, p8_ref, props_ref, ov_ref, tm_ref, mask_ref, f_ref, sf_ref):
    px1 = _clip(props_ref[0:1, :])
    py1 = _clip(props_ref[1:2, :])
    px2 = _clip(props_ref[2:3, :])
    py2 = _clip(props_ref[3:4, :])
    area = (px2 - px1 + 1.0) * (py2 - py1 + 1.0)

    ious = []
    for g in range(_G):
        gx1 = _clip(gt_ref[g, 0])
        gy1 = _clip(gt_ref[g, 1])
        gx2 = _clip(gt_ref[g, 2])
        gy2 = _clip(gt_ref[g, 3])
        iw = jnp.maximum(jnp.minimum(px2, gx2) - jnp.maximum(px1, gx1) + 1.0, 0.0)
        ih = jnp.maximum(jnp.minimum(py2, gy2) - jnp.maximum(py1, gy1) + 1.0, 0.0)
        inter = iw * ih
        ag = (gx2 - gx1 + 1.0) * (gy2 - gy1 + 1.0)
        ious.append(inter / (area + ag - inter))

    iota_c = jax.lax.broadcasted_iota(jnp.int32, (_NCLS, 1), 0)
    ov = jnp.zeros((_NCLS, _NP), jnp.float32)
    for g in range(_G):
        sel = iota_c == lab_ref[g]
        ov = jnp.maximum(ov, jnp.where(sel, ious[g], 0.0))
    ov_ref[...] = ov

    masks = []
    for g in range(_G):
        ol = ious[g]
        for g2 in range(_G):
            if g2 != g:
                same = lab_ref[g] == lab_ref[g2]
                ol = jnp.maximum(ol, jnp.where(same, ious[g2], 0.0))
        masks.append((ol > 0.6).astype(jnp.float32))
    mask = jnp.concatenate(masks, axis=0)
    mask_ref[...] = mask

    f = jnp.max(mask.reshape(_G, _NC, 16), axis=2)
    sf = jnp.max(f.reshape(_G, _NW, 16), axis=2)
    f_ref[...] = jnp.concatenate([f, jnp.zeros((_G, 384 - _NC), jnp.float32)], axis=1)
    sf_ref[...] = jnp.concatenate([sf, jnp.zeros((_G, 128 - _NW), jnp.float32)], axis=1)

    src_w = px2 - px1
    src_h = py2 - py1
    src_cx = px1 + 0.5 * src_w
    src_cy = py1 + 0.5 * src_h
    for g in range(_G):
        q1 = _clip(p8_ref[g, 0])
        q2 = _clip(p8_ref[g, 1])
        q3 = _clip(p8_ref[g, 2])
        q4 = _clip(p8_ref[g, 3])
        gw = q3 - q1
        gh = q4 - q2
        gcx = q1 + 0.5 * gw
        gcy = q2 + 0.5 * gh
        dcx = (gcx - src_cx) / src_w
        dcy = (gcy - src_cy) / src_h
        dw = jnp.log(gw / src_w)
        dh = jnp.log(gh / src_h)
        t4 = jnp.concatenate([dcx, dcy, dw, dh], axis=0) * masks[g]
        tm_ref[:, g, :] = t4


def _sc_body(mask_hbm, f_hbm, sf_hbm, x_hbm, out_hbm,
             m_v, f_v, sf_v, idx_v, rows_v, acc_v, cnt_s, sem):
    wid = lax.axis_index("s") * 2 + lax.axis_index("c")

    @pl.when(wid < _G)
    def _():
        g = wid
        lane = lax.broadcasted_iota(jnp.int32, (16,), 0)
        zeros16 = jnp.zeros((16,), jnp.float32)
        pltpu.sync_copy(mask_hbm.at[g], m_v)
        pltpu.sync_copy(f_hbm.at[g], f_v)
        pltpu.sync_copy(sf_hbm.at[g], sf_v)
        cnt_s[0] = 0

        def ext(vec, j):
            return vec[(lane + j) % 16][0]

        def append(n):
            w = cnt_s[0]
            base = (w // 16) * 16
            r = w - base

            @pl.when(r == 0)
            def _():
                idx_v[pl.ds(base, 16)] = jnp.where(lane == 0, n, 0)

            @pl.when(r > 0)
            def _():
                ivv = idx_v[pl.ds(base, 16)]
                idx_v[pl.ds(base, 16)] = jnp.where(lane == r, n, ivv)

            cnt_s[0] = w + 1

        sfa = sf_v[pl.ds(0, 16)]
        sfb = sf_v[pl.ds(16, 16)]

        def word_body(wb, carry):
            sflag = jnp.where(wb < 16, ext(sfa, wb % 16), ext(sfb, wb % 16))

            @pl.when(sflag > 0.5)
            def _():
                fw = f_v[pl.ds(wb * 16, 16)]

                def fj_body(fj, carry2):
                    @pl.when(ext(fw, fj) > 0.5)
                    def _():
                        c = wb * 16 + fj
                        mvec = m_v[pl.ds(c * 16, 16)]

                        def l_body(l, carry3):
                            @pl.when(ext(mvec, l) > 0.5)
                            def _():
                                append(c * 16 + l)

                            return carry3

                        lax.fori_loop(0, 16, l_body, 0)

                    return carry2

                lax.fori_loop(0, 16, fj_body, 0)

            return carry

        lax.fori_loop(0, _NW, word_body, 0)
        total = cnt_s[0]

        for k in range(_D // 16):
            acc_v[pl.ds(k * 16, 16)] = zeros16

        def batch_body(b, carry):
            pltpu.async_copy(x_hbm.at[idx_v.at[pl.ds(b * 16, 16)]], rows_v, sem).wait()
            rem = jnp.minimum(total - b * 16, 16)

            def row_body(l, carry2):
                for k in range(_D // 16):
                    acc_v[pl.ds(k * 16, 16)] += rows_v[l, pl.ds(k * 16, 16)]
                return carry2

            lax.fori_loop(0, rem, row_body, 0)
            return carry

        nb = (total + 15) // 16
        lax.fori_loop(0, nb, batch_body, 0)
        pltpu.sync_copy(acc_v, out_hbm.at[g])


def _posfeat_sc(mask_pad, flags, sflags, x):
    scmesh = plsc.VectorSubcoreMesh(core_axis_name="c", subcore_axis_name="s")
    f = functools.partial(
        pl.kernel,
        out_type=jax.ShapeDtypeStruct((_G, _D), jnp.float32),
        mesh=scmesh,
        scratch_types=[
            pltpu.VMEM((_NP,), jnp.float32),
            pltpu.VMEM((384,), jnp.float32),
            pltpu.VMEM((128,), jnp.float32),
            pltpu.VMEM((_NP + 16,), jnp.int32),
            pltpu.VMEM((16, _D), jnp.float32),
            pltpu.VMEM((_D,), jnp.float32),
            pltpu.SMEM((8,), jnp.int32),
            pltpu.SemaphoreType.DMA,
        ],
    )(_sc_body)
    return f(mask_pad, flags, sflags, x)


def kernel(x, proposals, gt_bbox, gt_labels):
    n, d = x.shape
    props_t = jnp.concatenate(
        [proposals.T, jnp.zeros((4, _NP - n), jnp.float32)], axis=1
    )
    p8 = proposals[:_G]

    ov_cn, tm, mask_pad, flags, sflags = pl.pallas_call(
        _head_body,
        in_specs=[
            pl.BlockSpec(memory_space=pltpu.SMEM),
            pl.BlockSpec(memory_space=pltpu.SMEM),
            pl.BlockSpec(memory_space=pltpu.SMEM),
            pl.BlockSpec((4, _NP), lambda: (0, 0)),
        ],
        out_specs=[
            pl.BlockSpec((_NCLS, _NP), lambda: (0, 0)),
            pl.BlockSpec((4, _G, _NP), lambda: (0, 0, 0)),
            pl.BlockSpec((_G, _NP), lambda: (0, 0)),
            pl.BlockSpec((_G, 384), lambda: (0, 0)),
            pl.BlockSpec((_G, 128), lambda: (0, 0)),
        ],
        out_shape=[
            jax.ShapeDtypeStruct((_NCLS, _NP), jnp.float32),
            jax.ShapeDtypeStruct((4, _G, _NP), jnp.float32),
            jax.ShapeDtypeStruct((_G, _NP), jnp.float32),
            jax.ShapeDtypeStruct((_G, 384), jnp.float32),
            jax.ShapeDtypeStruct((_G, 128), jnp.float32),
        ],
    )(gt_bbox, gt_labels.astype(jnp.int32), p8, props_t)

    pf = _posfeat_sc(mask_pad, flags, sflags, x)
    return ov_cn[:, :n].T, tm[:, :, :n].transpose(1, 2, 0), pf


# speedup vs baseline: 1.1761x; 1.0279x over previous
import functools

import jax
import jax.numpy as jnp
from jax import lax
from jax.experimental import pallas as pl
from jax.experimental.pallas import tpu as pltpu
from jax.experimental.pallas import tpu_sc as plsc

_NCLS = 30
_HI = 799.0
_G = 8
_NP = 5120
_D = 2048
_NC = _NP // 16
_NW = _NC // 16


def _clip(v):
    return jnp.clip(v, 1.0, _HI)


def _head_body(gt_ref, lab_ref, p8_ref, props_ref, ov_ref, tm_ref, comb_ref):
    px1 = _clip(props_ref[0:1, :])
    py1 = _clip(props_ref[1:2, :])
    px2 = _clip(props_ref[2:3, :])
    py2 = _clip(props_ref[3:4, :])
    area = (px2 - px1 + 1.0) * (py2 - py1 + 1.0)

    ious = []
    for g in range(_G):
        gx1 = _clip(gt_ref[g, 0])
        gy1 = _clip(gt_ref[g, 1])
        gx2 = _clip(gt_ref[g, 2])
        gy2 = _clip(gt_ref[g, 3])
        iw = jnp.maximum(jnp.minimum(px2, gx2) - jnp.maximum(px1, gx1) + 1.0, 0.0)
        ih = jnp.maximum(jnp.minimum(py2, gy2) - jnp.maximum(py1, gy1) + 1.0, 0.0)
        inter = iw * ih
        ag = (gx2 - gx1 + 1.0) * (gy2 - gy1 + 1.0)
        ious.append(inter / (area + ag - inter))

    iota_c = jax.lax.broadcasted_iota(jnp.int32, (_NCLS, 1), 0)
    ov = jnp.zeros((_NCLS, _NP), jnp.float32)
    for g in range(_G):
        sel = iota_c == lab_ref[g]
        ov = jnp.maximum(ov, jnp.where(sel, ious[g], 0.0))
    ov_ref[...] = ov

    masks = []
    for g in range(_G):
        ol = ious[g]
        for g2 in range(_G):
            if g2 != g:
                same = lab_ref[g] == lab_ref[g2]
                ol = jnp.maximum(ol, jnp.where(same, ious[g2], 0.0))
        masks.append((ol > 0.6).astype(jnp.float32))
    mask = jnp.concatenate(masks, axis=0)

    f = jnp.max(mask.reshape(_G, _NC, 16), axis=2)
    sf = jnp.max(f.reshape(_G, _NW, 16), axis=2)
    comb_ref[...] = jnp.concatenate(
        [mask,
         f, jnp.zeros((_G, 384 - _NC), jnp.float32),
         sf, jnp.zeros((_G, 128 - _NW), jnp.float32)], axis=1)

    src_w = px2 - px1
    src_h = py2 - py1
    src_cx = px1 + 0.5 * src_w
    src_cy = py1 + 0.5 * src_h
    for g in range(_G):
        q1 = _clip(p8_ref[g, 0])
        q2 = _clip(p8_ref[g, 1])
        q3 = _clip(p8_ref[g, 2])
        q4 = _clip(p8_ref[g, 3])
        gw = q3 - q1
        gh = q4 - q2
        gcx = q1 + 0.5 * gw
        gcy = q2 + 0.5 * gh
        dcx = (gcx - src_cx) / src_w
        dcy = (gcy - src_cy) / src_h
        dw = jnp.log(gw / src_w)
        dh = jnp.log(gh / src_h)
        t4 = jnp.concatenate([dcx, dcy, dw, dh], axis=0) * masks[g]
        tm_ref[:, g, :] = t4


def _sc_body(comb_hbm, x_hbm, out_hbm, m_v, idx_v, rows_v, acc_v, cnt_s, sem):
    wid = lax.axis_index("s")
    cid = lax.axis_index("c")

    @pl.when(jnp.logical_and(cid == 0, wid < _G))
    def _():
        g = wid
        lane = lax.broadcasted_iota(jnp.int32, (16,), 0)
        zeros16 = jnp.zeros((16,), jnp.float32)
        pltpu.sync_copy(comb_hbm.at[g], m_v)
        cnt_s[0] = 0

        def ext(vec, j):
            return vec[(lane + j) % 16][0]

        def append(n):
            w = cnt_s[0]
            base = (w // 16) * 16
            r = w - base

            @pl.when(r == 0)
            def _():
                idx_v[pl.ds(base, 16)] = jnp.where(lane == 0, n, 0)

            @pl.when(r > 0)
            def _():
                ivv = idx_v[pl.ds(base, 16)]
                idx_v[pl.ds(base, 16)] = jnp.where(lane == r, n, ivv)

            cnt_s[0] = w + 1

        sfa = m_v[pl.ds(_NP + 384, 16)]
        sfb = m_v[pl.ds(_NP + 384 + 16, 16)]

        def word_body(wb, carry):
            sflag = jnp.where(wb < 16, ext(sfa, wb % 16), ext(sfb, wb % 16))

            @pl.when(sflag > 0.5)
            def _():
                fw = m_v[pl.ds(_NP + wb * 16, 16)]

                def fj_body(fj, carry2):
                    @pl.when(ext(fw, fj) > 0.5)
                    def _():
                        c = wb * 16 + fj
                        mvec = m_v[pl.ds(c * 16, 16)]

                        def l_body(l, carry3):
                            @pl.when(ext(mvec, l) > 0.5)
                            def _():
                                append(c * 16 + l)

                            return carry3

                        lax.fori_loop(0, 16, l_body, 0)

                    return carry2

                lax.fori_loop(0, 16, fj_body, 0)

            return carry

        lax.fori_loop(0, _NW, word_body, 0)
        total = cnt_s[0]

        for k in range(_D // 16):
            acc_v[pl.ds(k * 16, 16)] = zeros16

        def batch_body(b, carry):
            pltpu.async_copy(x_hbm.at[idx_v.at[pl.ds(b * 16, 16)]], rows_v, sem).wait()
            rem = jnp.minimum(total - b * 16, 16)

            def row_body(l, carry2):
                for k in range(_D // 16):
                    acc_v[pl.ds(k * 16, 16)] += rows_v[l, pl.ds(k * 16, 16)]
                return carry2

            lax.fori_loop(0, rem, row_body, 0)
            return carry

        nb = (total + 15) // 16
        lax.fori_loop(0, nb, batch_body, 0)
        pltpu.sync_copy(acc_v, out_hbm.at[g])


def _posfeat_sc(comb, x):
    scmesh = plsc.VectorSubcoreMesh(core_axis_name="c", subcore_axis_name="s")
    f = functools.partial(
        pl.kernel,
        out_type=jax.ShapeDtypeStruct((_G, _D), jnp.float32),
        mesh=scmesh,
        scratch_types=[
            pltpu.VMEM((_NP + 512,), jnp.float32),
            pltpu.VMEM((_NP + 16,), jnp.int32),
            pltpu.VMEM((16, _D), jnp.float32),
            pltpu.VMEM((_D,), jnp.float32),
            pltpu.SMEM((8,), jnp.int32),
            pltpu.SemaphoreType.DMA,
        ],
    )(_sc_body)
    return f(comb, x)


def kernel(x, proposals, gt_bbox, gt_labels):
    n, d = x.shape
    props_t = jnp.concatenate(
        [proposals.T, jnp.zeros((4, _NP - n), jnp.float32)], axis=1
    )
    p8 = proposals[:_G]

    ov_cn, tm, comb = pl.pallas_call(
        _head_body,
        in_specs=[
            pl.BlockSpec(memory_space=pltpu.SMEM),
            pl.BlockSpec(memory_space=pltpu.SMEM),
            pl.BlockSpec(memory_space=pltpu.SMEM),
            pl.BlockSpec((4, _NP), lambda: (0, 0)),
        ],
        out_specs=[
            pl.BlockSpec((_NCLS, _NP), lambda: (0, 0)),
            pl.BlockSpec((4, _G, _NP), lambda: (0, 0, 0)),
            pl.BlockSpec((_G, _NP + 512), lambda: (0, 0)),
        ],
        out_shape=[
            jax.ShapeDtypeStruct((_NCLS, _NP), jnp.float32),
            jax.ShapeDtypeStruct((4, _G, _NP), jnp.float32),
            jax.ShapeDtypeStruct((_G, _NP + 512), jnp.float32),
        ],
    )(gt_bbox, gt_labels.astype(jnp.int32), p8, props_t)

    pf = _posfeat_sc(comb, x)
    return ov_cn[:, :n].T, tm[:, :, :n].transpose(1, 2, 0), pf
